# baseline (speedup 1.0000x reference)
import jax
import jax.numpy as jnp
import numpy as np
from jax import lax
from jax.experimental import pallas as pl
from jax.experimental.pallas import tpu as pltpu
from jax.experimental.pallas import tpu_sc as plsc

_P = 0.9
_B, _C, _H, _W = 4, 96, 384, 384
_N = _H * _W
_BC = _B * _C

_PAD = 3072
_L = 18432
_NCH = _N // _L
_WWIN = _L + 2 * _PAD

_NC, _NS = 2, 16
_NW = _NC * _NS
_RPW = _BC // _NW
_U = 8
_VPC = _L // 16


def _perm_fixed(n_element, dim, p):
    probs = np.array([1.0 - p / 2.0, p / 8.0, p / 8.0, p / 8.0, p / 8.0])
    rng = np.random.default_rng(0)
    rs = rng.choice(5, size=2 * n_element - 1, p=probs)
    perm = np.arange(n_element, dtype=np.int64)
    t = 0
    for i in range(-n_element + 1, n_element):
        r = rs[t]
        t += 1
        i = abs(i)
        if r != 0:
            if r == 1:
                idx = i + 1
            elif r == 2:
                idx = i - 1
            elif r == 3:
                idx = i + dim
            else:
                idx = i - dim
            if 0 < idx < n_element:
                tmp = int(perm[i])
                perm[i] = int(perm[idx])
                perm[idx] = tmp
    return perm


def _local_indices():
    perm = _perm_fixed(_N, _W, _P)
    li = np.empty(_N, dtype=np.int32)
    for ch in range(_NCH):
        j0 = ch * _L
        w0 = min(max(j0 - _PAD, 0), _N - _WWIN)
        seg = perm[j0:j0 + _L] - w0
        assert seg.min() >= 0 and seg.max() < _WWIN
        li[j0:j0 + _L] = seg.astype(np.int32)
    return li


_LI_NP = _local_indices()


def _body(x_hbm, li_hbm, o_hbm, idx_v, win_v, out_v):
    cid = lax.axis_index("c")
    sid = lax.axis_index("s")
    wid = sid * _NC + cid
    row0 = wid * _RPW

    def chunk_body(ch, carry):
        j0 = pl.multiple_of(ch * _L, _L)
        w0 = pl.multiple_of(jnp.clip(j0 - _PAD, 0, _N - _WWIN), _PAD)
        pltpu.sync_copy(li_hbm.at[pl.ds(j0, _L)], idx_v)

        def row_body(r, carry2):
            row = row0 + r
            src = pl.multiple_of(row * _N + w0, 1024)
            pltpu.sync_copy(x_hbm.at[pl.ds(src, _WWIN)], win_v)

            def g_body(v, carry3):
                base = v * (16 * _U)
                for u in range(_U):
                    iv = idx_v[pl.ds(base + u * 16, 16)]
                    vals = plsc.load_gather(win_v, [iv])
                    out_v[pl.ds(base + u * 16, 16)] = vals
                return carry3

            lax.fori_loop(0, _VPC // _U, g_body, 0)
            dst = pl.multiple_of(row * _N + j0, 1024)
            pltpu.sync_copy(out_v, o_hbm.at[pl.ds(dst, _L)])
            return carry2

        lax.fori_loop(0, _RPW, row_body, 0)
        return carry

    lax.fori_loop(0, _NCH, chunk_body, 0)


@jax.jit
def _run(x2, li):
    mesh = plsc.VectorSubcoreMesh(core_axis_name="c", subcore_axis_name="s")
    f = pl.kernel(
        _body,
        mesh=mesh,
        compiler_params=pltpu.CompilerParams(needs_layout_passes=False),
        out_type=jax.ShapeDtypeStruct((_BC * _N,), jnp.float32),
        scratch_types=[
            pltpu.VMEM((_L,), jnp.int32),
            pltpu.VMEM((_WWIN,), jnp.float32),
            pltpu.VMEM((_L,), jnp.float32),
        ],
    )
    return f(x2, li)


def kernel(x):
    x1 = x.reshape(_BC * _N)
    out1 = _run(x1, jnp.asarray(_LI_NP))
    return out1.reshape(x.shape)


# speedup vs baseline: 1.2194x; 1.2194x over previous
import jax
import jax.numpy as jnp
import numpy as np
from jax import lax
from jax.experimental import pallas as pl
from jax.experimental.pallas import tpu as pltpu
from jax.experimental.pallas import tpu_sc as plsc

_P = 0.9
_B, _C, _H, _W = 4, 96, 384, 384
_N = _H * _W
_BC = _B * _C

_PAD = 3072
_L = 18432
_NCH = _N // _L
_WWIN = _L + 2 * _PAD

_NC, _NS = 2, 16
_NW = _NC * _NS
_RPW = _BC // _NW
_U = 8
_VPC = _L // 16


def _perm_fixed(n_element, dim, p):
    probs = np.array([1.0 - p / 2.0, p / 8.0, p / 8.0, p / 8.0, p / 8.0])
    rng = np.random.default_rng(0)
    rs = rng.choice(5, size=2 * n_element - 1, p=probs)
    perm = np.arange(n_element, dtype=np.int64)
    t = 0
    for i in range(-n_element + 1, n_element):
        r = rs[t]
        t += 1
        i = abs(i)
        if r != 0:
            if r == 1:
                idx = i + 1
            elif r == 2:
                idx = i - 1
            elif r == 3:
                idx = i + dim
            else:
                idx = i - dim
            if 0 < idx < n_element:
                tmp = int(perm[i])
                perm[i] = int(perm[idx])
                perm[idx] = tmp
    return perm


def _local_indices():
    perm = _perm_fixed(_N, _W, _P)
    li = np.empty(_N, dtype=np.int32)
    for ch in range(_NCH):
        j0 = ch * _L
        w0 = min(max(j0 - _PAD, 0), _N - _WWIN)
        seg = perm[j0:j0 + _L] - w0
        assert seg.min() >= 0 and seg.max() < _WWIN
        li[j0:j0 + _L] = seg.astype(np.int32)
    return li


_LI_NP = _local_indices()


def _body(x_hbm, li_hbm, o_hbm, idx_v, win0_v, win1_v, out0_v, out1_v,
          wsem0, wsem1, osem0, osem1):
    cid = lax.axis_index("c")
    sid = lax.axis_index("s")
    wid = sid * _NC + cid
    row0 = wid * _RPW

    wins = [win0_v, win1_v]
    outs = [out0_v, out1_v]
    wsems = [wsem0, wsem1]
    osems = [osem0, osem1]

    def chunk_body(ch, carry):
        j0 = pl.multiple_of(ch * _L, _L)
        w0 = pl.multiple_of(jnp.clip(j0 - _PAD, 0, _N - _WWIN), _PAD)
        pltpu.sync_copy(li_hbm.at[pl.ds(j0, _L)], idx_v)

        def win_start(r):
            src = pl.multiple_of((row0 + r) * _N + w0, 1024)
            return pltpu.async_copy(
                x_hbm.at[pl.ds(src, _WWIN)], wins[r % 2], wsems[r % 2])

        win_copies = {0: win_start(0)}
        out_copies = {}
        for r in range(_RPW):
            if r + 1 < _RPW:
                win_copies[r + 1] = win_start(r + 1)
            win_copies[r].wait()
            if r >= 2:
                out_copies[r - 2].wait()
            wv = wins[r % 2]
            ov = outs[r % 2]

            def g_body(v, carry3, wv=wv, ov=ov):
                base = v * (16 * _U)
                for u in range(_U):
                    iv = idx_v[pl.ds(base + u * 16, 16)]
                    ov[pl.ds(base + u * 16, 16)] = plsc.load_gather(wv, [iv])
                return carry3

            lax.fori_loop(0, _VPC // _U, g_body, 0)
            dst = pl.multiple_of((row0 + r) * _N + j0, 1024)
            out_copies[r] = pltpu.async_copy(
                ov, o_hbm.at[pl.ds(dst, _L)], osems[r % 2])
        out_copies[_RPW - 2].wait()
        out_copies[_RPW - 1].wait()
        return carry

    lax.fori_loop(0, _NCH, chunk_body, 0)


@jax.jit
def _run(x2, li):
    mesh = plsc.VectorSubcoreMesh(core_axis_name="c", subcore_axis_name="s")
    f = pl.kernel(
        _body,
        mesh=mesh,
        compiler_params=pltpu.CompilerParams(needs_layout_passes=False),
        out_type=jax.ShapeDtypeStruct((_BC * _N,), jnp.float32),
        scratch_types=[
            pltpu.VMEM((_L,), jnp.int32),
            pltpu.VMEM((_WWIN,), jnp.float32),
            pltpu.VMEM((_WWIN,), jnp.float32),
            pltpu.VMEM((_L,), jnp.float32),
            pltpu.VMEM((_L,), jnp.float32),
            pltpu.SemaphoreType.DMA,
            pltpu.SemaphoreType.DMA,
            pltpu.SemaphoreType.DMA,
            pltpu.SemaphoreType.DMA,
        ],
    )
    return f(x2, li)


def kernel(x):
    x1 = x.reshape(_BC * _N)
    out1 = _run(x1, jnp.asarray(_LI_NP))
    return out1.reshape(x.shape)


# speedup vs baseline: 1.7260x; 1.4154x over previous
import jax
import jax.numpy as jnp
import numpy as np
from jax import lax
from jax.experimental import pallas as pl
from jax.experimental.pallas import tpu as pltpu
from jax.experimental.pallas import tpu_sc as plsc

_P = 0.9
_B, _C, _H, _W = 4, 96, 384, 384
_N = _H * _W
_BC = _B * _C

_PAD = 3072
_L = 18432
_NCH = _N // _L
_WWIN = _L + 2 * _PAD

_NC, _NS = 2, 16
_NW = _NC * _NS
_RPW = _BC // _NW
_U = 8
_VPC = _L // 16


def _perm_fixed(n_element, dim, p):
    probs = np.array([1.0 - p / 2.0, p / 8.0, p / 8.0, p / 8.0, p / 8.0])
    rng = np.random.default_rng(0)
    rs = rng.choice(5, size=2 * n_element - 1, p=probs)
    perm = np.arange(n_element, dtype=np.int64)
    t = 0
    for i in range(-n_element + 1, n_element):
        r = rs[t]
        t += 1
        i = abs(i)
        if r != 0:
            if r == 1:
                idx = i + 1
            elif r == 2:
                idx = i - 1
            elif r == 3:
                idx = i + dim
            else:
                idx = i - dim
            if 0 < idx < n_element:
                tmp = int(perm[i])
                perm[i] = int(perm[idx])
                perm[idx] = tmp
    return perm


def _local_indices():
    perm = _perm_fixed(_N, _W, _P)
    li = np.empty(_N, dtype=np.int32)
    for ch in range(_NCH):
        j0 = ch * _L
        w0 = min(max(j0 - _PAD, 0), _N - _WWIN)
        seg = perm[j0:j0 + _L] - w0
        assert seg.min() >= 0 and seg.max() < _WWIN
        li[j0:j0 + _L] = seg.astype(np.int32)
    return li


_LI_NP = _local_indices()


def _body(x_hbm, li_hbm, o_hbm, idx_v, win0_v, win1_v, out0_v, out1_v,
          wsem0, wsem1, osem0, osem1):
    cid = lax.axis_index("c")
    sid = lax.axis_index("s")
    wid = sid * _NC + cid
    row0 = wid * _RPW

    wins = [win0_v, win1_v]
    outs = [out0_v, out1_v]
    wsems = [wsem0, wsem1]
    osems = [osem0, osem1]

    def chunk_body(ch, carry):
        j0 = pl.multiple_of(ch * _L, _L)
        w0 = pl.multiple_of(jnp.clip(j0 - _PAD, 0, _N - _WWIN), _PAD)
        pltpu.sync_copy(li_hbm.at[pl.ds(j0, _L)], idx_v)

        def win_start(r):
            src = pl.multiple_of((row0 + r) * _N + w0, 1024)
            return pltpu.async_copy(
                x_hbm.at[pl.ds(src, _WWIN)], wins[r % 2], wsems[r % 2])

        win_copies = {0: win_start(0)}
        out_copies = {}
        for r in range(_RPW):
            if r + 1 < _RPW:
                win_copies[r + 1] = win_start(r + 1)
            win_copies[r].wait()
            if r >= 2:
                out_copies[r - 2].wait()
            wv = wins[r % 2]
            ov = outs[r % 2]

            @plsc.parallel_loop(0, _VPC, unroll=_U)
            def g_body(v, wv=wv, ov=ov):
                iv = idx_v[pl.ds(v * 16, 16)]
                ov[pl.ds(v * 16, 16)] = plsc.load_gather(wv, [iv])
            dst = pl.multiple_of((row0 + r) * _N + j0, 1024)
            out_copies[r] = pltpu.async_copy(
                ov, o_hbm.at[pl.ds(dst, _L)], osems[r % 2])
        out_copies[_RPW - 2].wait()
        out_copies[_RPW - 1].wait()
        return carry

    lax.fori_loop(0, _NCH, chunk_body, 0)


@jax.jit
def _run(x2, li):
    mesh = plsc.VectorSubcoreMesh(core_axis_name="c", subcore_axis_name="s")
    f = pl.kernel(
        _body,
        mesh=mesh,
        compiler_params=pltpu.CompilerParams(needs_layout_passes=False),
        out_type=jax.ShapeDtypeStruct((_BC * _N,), jnp.float32),
        scratch_types=[
            pltpu.VMEM((_L,), jnp.int32),
            pltpu.VMEM((_WWIN,), jnp.float32),
            pltpu.VMEM((_WWIN,), jnp.float32),
            pltpu.VMEM((_L,), jnp.float32),
            pltpu.VMEM((_L,), jnp.float32),
            pltpu.SemaphoreType.DMA,
            pltpu.SemaphoreType.DMA,
            pltpu.SemaphoreType.DMA,
            pltpu.SemaphoreType.DMA,
        ],
    )
    return f(x2, li)


def kernel(x):
    x1 = x.reshape(_BC * _N)
    out1 = _run(x1, jnp.asarray(_LI_NP))
    return out1.reshape(x.shape)


# speedup vs baseline: 3.0950x; 1.7932x over previous
import jax
import jax.numpy as jnp
import numpy as np
from jax import lax
from jax.experimental import pallas as pl
from jax.experimental.pallas import tpu as pltpu
from jax.experimental.pallas import tpu_sc as plsc

_P = 0.9
_B, _C, _H, _W = 4, 96, 384, 384
_N = _H * _W
_BC = _B * _C

_HPAD = 8
_HCH = 48
_NCH = _H // _HCH
_HWIN = _HCH + 2 * _HPAD
_L = _HCH * _W
_WWIN = _HWIN * _W

_NC, _NS = 2, 16
_NW = _NC * _NS
_PPW = _BC // _NW
_U = 8
_GPR = _W // 16


def _perm_fixed(n_element, dim, p):
    probs = np.array([1.0 - p / 2.0, p / 8.0, p / 8.0, p / 8.0, p / 8.0])
    rng = np.random.default_rng(0)
    rs = rng.choice(5, size=2 * n_element - 1, p=probs)
    perm = np.arange(n_element, dtype=np.int64)
    t = 0
    for i in range(-n_element + 1, n_element):
        r = rs[t]
        t += 1
        i = abs(i)
        if r != 0:
            if r == 1:
                idx = i + 1
            elif r == 2:
                idx = i - 1
            elif r == 3:
                idx = i + dim
            else:
                idx = i - dim
            if 0 < idx < n_element:
                tmp = int(perm[i])
                perm[i] = int(perm[idx])
                perm[idx] = tmp
    return perm


def _local_indices():
    perm = _perm_fixed(_N, _W, _P)
    li = np.empty(_N, dtype=np.int32)
    for ch in range(_NCH):
        j0 = ch * _L
        w0 = min(max(j0 - _HPAD * _W, 0), _N - _WWIN)
        seg = perm[j0:j0 + _L] - w0
        assert seg.min() >= 0 and seg.max() < _WWIN
        li[j0:j0 + _L] = ((seg // _W) * 512 + seg % _W).astype(np.int32)
    return li


_LI_NP = _local_indices()


def _body(x_hbm, li_hbm, o_hbm, idx_v, win0_v, win1_v, out0_v, out1_v,
          wsem0, wsem1, osem0, osem1):
    cid = lax.axis_index("c")
    sid = lax.axis_index("s")
    wid = sid * _NC + cid
    p0 = wid * _PPW

    wins = [win0_v, win1_v]
    outs = [out0_v, out1_v]
    wsems = [wsem0, wsem1]
    osems = [osem0, osem1]

    def chunk_body(ch, carry):
        hj0 = pl.multiple_of(ch * _HCH, _HCH)
        h0 = pl.multiple_of(jnp.clip(hj0 - _HPAD, 0, _H - _HWIN), _HPAD)
        pltpu.sync_copy(li_hbm.at[pl.ds(pl.multiple_of(ch * _L, _L), _L)],
                        idx_v)

        def plane_bc(r):
            p = p0 + r
            return p // _C, p % _C

        def win_start(r):
            b, c = plane_bc(r)
            return pltpu.async_copy(
                x_hbm.at[b, c, pl.ds(h0, _HWIN), :],
                wins[r % 2], wsems[r % 2])

        win_copies = {0: win_start(0)}
        out_copies = {}
        for r in range(_PPW):
            if r + 1 < _PPW:
                win_copies[r + 1] = win_start(r + 1)
            win_copies[r].wait()
            if r >= 2:
                out_copies[r - 2].wait()
            wv = wins[r % 2]
            ov = outs[r % 2]

            @plsc.parallel_loop(0, _HCH)
            def row_body(rr, wv=wv, ov=ov):
                rbase = rr * _W

                @plsc.parallel_loop(0, _GPR, unroll=_U)
                def g_body(vv, rr=rr, rbase=rbase, wv=wv, ov=ov):
                    iv = idx_v[pl.ds(rbase + vv * 16, 16)]
                    ri = iv >> 9
                    ci = iv & 511
                    ov[rr, pl.ds(vv * 16, 16)] = plsc.load_gather(
                        wv, [ri, ci])

            b, c = plane_bc(r)
            out_copies[r] = pltpu.async_copy(
                ov, o_hbm.at[b, c, pl.ds(hj0, _HCH), :], osems[r % 2])
        out_copies[_PPW - 2].wait()
        out_copies[_PPW - 1].wait()
        return carry

    lax.fori_loop(0, _NCH, chunk_body, 0)


@jax.jit
def _run(x, li):
    mesh = plsc.VectorSubcoreMesh(core_axis_name="c", subcore_axis_name="s")
    f = pl.kernel(
        _body,
        mesh=mesh,
        compiler_params=pltpu.CompilerParams(needs_layout_passes=False),
        out_type=jax.ShapeDtypeStruct((_B, _C, _H, _W), jnp.float32),
        scratch_types=[
            pltpu.VMEM((_L,), jnp.int32),
            pltpu.VMEM((_HWIN, _W), jnp.float32),
            pltpu.VMEM((_HWIN, _W), jnp.float32),
            pltpu.VMEM((_HCH, _W), jnp.float32),
            pltpu.VMEM((_HCH, _W), jnp.float32),
            pltpu.SemaphoreType.DMA,
            pltpu.SemaphoreType.DMA,
            pltpu.SemaphoreType.DMA,
            pltpu.SemaphoreType.DMA,
        ],
    )
    return f(x, li)


def kernel(x):
    return _run(x, jnp.asarray(_LI_NP))


# speedup vs baseline: 3.3187x; 1.0723x over previous
import jax
import jax.numpy as jnp
import numpy as np
from jax import lax
from jax.experimental import pallas as pl
from jax.experimental.pallas import tpu as pltpu
from jax.experimental.pallas import tpu_sc as plsc

_P = 0.9
_B, _C, _H, _W = 4, 96, 384, 384
_N = _H * _W
_BC = _B * _C

_HPAD = 8
_HCH = 48
_NCH = _H // _HCH
_HWIN = _HCH + 2 * _HPAD
_L = _HCH * _W
_WWIN = _HWIN * _W

_NC, _NS = 2, 16
_NW = _NC * _NS
_PPW = _BC // _NW
_U = 8
_GPR = _W // 16


def _perm_fixed(n_element, dim, p):
    probs = np.array([1.0 - p / 2.0, p / 8.0, p / 8.0, p / 8.0, p / 8.0])
    rng = np.random.default_rng(0)
    rs = rng.choice(5, size=2 * n_element - 1, p=probs)
    perm = np.arange(n_element, dtype=np.int64)
    t = 0
    for i in range(-n_element + 1, n_element):
        r = rs[t]
        t += 1
        i = abs(i)
        if r != 0:
            if r == 1:
                idx = i + 1
            elif r == 2:
                idx = i - 1
            elif r == 3:
                idx = i + dim
            else:
                idx = i - dim
            if 0 < idx < n_element:
                tmp = int(perm[i])
                perm[i] = int(perm[idx])
                perm[idx] = tmp
    return perm


def _local_indices():
    perm = _perm_fixed(_N, _W, _P)
    li = np.empty(_N, dtype=np.int32)
    for ch in range(_NCH):
        j0 = ch * _L
        w0 = min(max(j0 - _HPAD * _W, 0), _N - _WWIN)
        seg = perm[j0:j0 + _L] - w0
        assert seg.min() >= 0 and seg.max() < _WWIN
        packed = ((seg // _W) * 512 + seg % _W).astype(np.int32)
        packed = packed.reshape(_HCH, _GPR, 16).transpose(1, 0, 2)
        li[j0:j0 + _L] = packed.reshape(-1)
    return li


_LI_NP = _local_indices()


def _body(x_hbm, li_hbm, o_hbm, idx_v, win0_v, win1_v, out0_v, out1_v,
          wsem0, wsem1, osem0, osem1):
    cid = lax.axis_index("c")
    sid = lax.axis_index("s")
    wid = sid * _NC + cid
    p0 = wid * _PPW

    wins = [win0_v, win1_v]
    outs = [out0_v, out1_v]
    wsems = [wsem0, wsem1]
    osems = [osem0, osem1]

    def chunk_body(ch, carry):
        hj0 = pl.multiple_of(ch * _HCH, _HCH)
        h0 = pl.multiple_of(jnp.clip(hj0 - _HPAD, 0, _H - _HWIN), _HPAD)
        pltpu.sync_copy(li_hbm.at[pl.ds(pl.multiple_of(ch * _L, _L), _L)],
                        idx_v)

        def plane_bc(r):
            p = p0 + r
            return p // _C, p % _C

        def win_start(r):
            b, c = plane_bc(r)
            return pltpu.async_copy(
                x_hbm.at[b, c, pl.ds(h0, _HWIN), :],
                wins[r % 2], wsems[r % 2])

        win_copies = {0: win_start(0)}
        out_copies = {}
        for r in range(_PPW):
            if r + 1 < _PPW:
                win_copies[r + 1] = win_start(r + 1)
            win_copies[r].wait()
            if r >= 2:
                out_copies[r - 2].wait()
            wv = wins[r % 2]
            ov = outs[r % 2]

            @plsc.parallel_loop(0, _GPR)
            def col_body(cc, wv=wv, ov=ov):
                cbase = cc * (_HCH * 16)
                c16 = cc * 16

                @plsc.parallel_loop(0, _HCH, unroll=_U)
                def row_body(rr, cbase=cbase, c16=c16, wv=wv, ov=ov):
                    iv = idx_v[pl.ds(cbase + rr * 16, 16)]
                    ri = iv >> 9
                    ci = iv & 511
                    ov[rr, pl.ds(c16, 16)] = plsc.load_gather(wv, [ri, ci])

            b, c = plane_bc(r)
            out_copies[r] = pltpu.async_copy(
                outs[r % 2], o_hbm.at[b, c, pl.ds(hj0, _HCH), :],
                osems[r % 2])
        out_copies[_PPW - 2].wait()
        out_copies[_PPW - 1].wait()
        return carry

    lax.fori_loop(0, _NCH, chunk_body, 0)


@jax.jit
def _run(x, li):
    mesh = plsc.VectorSubcoreMesh(core_axis_name="c", subcore_axis_name="s")
    f = pl.kernel(
        _body,
        mesh=mesh,
        compiler_params=pltpu.CompilerParams(needs_layout_passes=False),
        out_type=jax.ShapeDtypeStruct((_B, _C, _H, _W), jnp.float32),
        scratch_types=[
            pltpu.VMEM((_L,), jnp.int32),
            pltpu.VMEM((_HWIN, _W), jnp.float32),
            pltpu.VMEM((_HWIN, _W), jnp.float32),
            pltpu.VMEM((_HCH, _W), jnp.float32),
            pltpu.VMEM((_HCH, _W), jnp.float32),
            pltpu.SemaphoreType.DMA,
            pltpu.SemaphoreType.DMA,
            pltpu.SemaphoreType.DMA,
            pltpu.SemaphoreType.DMA,
        ],
    )
    return f(x, li)


def kernel(x):
    return _run(x, jnp.asarray(_LI_NP))


# speedup vs baseline: 4.3161x; 1.3005x over previous
import jax
import jax.numpy as jnp
import numpy as np
from jax import lax
from jax.experimental import pallas as pl
from jax.experimental.pallas import tpu as pltpu
from jax.experimental.pallas import tpu_sc as plsc

_P = 0.9
_B, _C, _H, _W = 4, 96, 384, 384
_N = _H * _W
_BC = _B * _C

_HPAD = 8
_HCH = 48
_NCH = _H // _HCH
_HWIN = _HCH + 2 * _HPAD
_L = _HCH * _W
_WWIN = _HWIN * _W

_NC, _NS = 2, 16
_NW = _NC * _NS
_PPW = _BC // _NW
_U = 8
_GPR = _W // 16


def _perm_fixed(n_element, dim, p):
    probs = np.array([1.0 - p / 2.0, p / 8.0, p / 8.0, p / 8.0, p / 8.0])
    rng = np.random.default_rng(0)
    rs = rng.choice(5, size=2 * n_element - 1, p=probs)
    perm = np.arange(n_element, dtype=np.int64)
    t = 0
    for i in range(-n_element + 1, n_element):
        r = rs[t]
        t += 1
        i = abs(i)
        if r != 0:
            if r == 1:
                idx = i + 1
            elif r == 2:
                idx = i - 1
            elif r == 3:
                idx = i + dim
            else:
                idx = i - dim
            if 0 < idx < n_element:
                tmp = int(perm[i])
                perm[i] = int(perm[idx])
                perm[idx] = tmp
    return perm


def _local_indices():
    perm = _perm_fixed(_N, _W, _P)
    li = np.empty(_N, dtype=np.int32)
    for ch in range(_NCH):
        j0 = ch * _L
        w0 = min(max(j0 - _HPAD * _W, 0), _N - _WWIN)
        seg = perm[j0:j0 + _L] - w0
        assert seg.min() >= 0 and seg.max() < _WWIN
        packed = ((seg // _W) * 512 + seg % _W).astype(np.int32)
        packed = packed.reshape(_HCH, _GPR, 16).transpose(1, 0, 2)
        li[j0:j0 + _L] = packed.reshape(-1)
    return li


_LI_NP = _local_indices()


def _body(x_hbm, li_hbm, o_hbm, idx_v, win0_v, win1_v, out0_v, out1_v,
          wsem0, wsem1, osem0, osem1):
    cid = lax.axis_index("c")
    sid = lax.axis_index("s")
    wid = sid * _NC + cid
    p0 = wid * _PPW

    wins = [win0_v, win1_v]
    outs = [out0_v, out1_v]
    wsems = [wsem0, wsem1]
    osems = [osem0, osem1]

    def chunk_body(ch, carry):
        hj0 = pl.multiple_of(ch * _HCH, _HCH)
        h0 = pl.multiple_of(jnp.clip(hj0 - _HPAD, 0, _H - _HWIN), _HPAD)
        pltpu.sync_copy(li_hbm.at[pl.ds(pl.multiple_of(ch * _L, _L), _L)],
                        idx_v)

        def plane_bc(r):
            p = p0 + r
            return p // _C, p % _C

        def win_start(r):
            b, c = plane_bc(r)
            return pltpu.async_copy(
                x_hbm.at[b, c, pl.ds(h0, _HWIN), :],
                wins[r % 2].at[:, pl.ds(0, _W)], wsems[r % 2])

        win_copies = {0: win_start(0)}
        out_copies = {}
        for r in range(_PPW):
            if r + 1 < _PPW:
                win_copies[r + 1] = win_start(r + 1)
            win_copies[r].wait()
            if r >= 2:
                out_copies[r - 2].wait()
            wv = wins[r % 2]
            ov = outs[r % 2]

            @plsc.parallel_loop(0, _GPR)
            def col_body(cc, wv=wv, ov=ov):
                cbase = cc * (_HCH * 16)
                c16 = cc * 16

                @plsc.parallel_loop(0, _HCH, unroll=_U)
                def row_body(rr, cbase=cbase, c16=c16, wv=wv, ov=ov):
                    iv = idx_v[pl.ds(cbase + rr * 16, 16)]
                    ri = iv >> 9
                    ci = iv & 511
                    ov[rr, pl.ds(c16, 16)] = plsc.load_gather(wv, [ri, ci])

            b, c = plane_bc(r)
            out_copies[r] = pltpu.async_copy(
                outs[r % 2], o_hbm.at[b, c, pl.ds(hj0, _HCH), :],
                osems[r % 2])
        out_copies[_PPW - 2].wait()
        out_copies[_PPW - 1].wait()
        return carry

    lax.fori_loop(0, _NCH, chunk_body, 0)


@jax.jit
def _run(x, li):
    mesh = plsc.VectorSubcoreMesh(core_axis_name="c", subcore_axis_name="s")
    f = pl.kernel(
        _body,
        mesh=mesh,
        compiler_params=pltpu.CompilerParams(needs_layout_passes=False),
        out_type=jax.ShapeDtypeStruct((_B, _C, _H, _W), jnp.float32),
        scratch_types=[
            pltpu.VMEM((_L,), jnp.int32),
            pltpu.VMEM((_HWIN, 512), jnp.float32),
            pltpu.VMEM((_HWIN, 512), jnp.float32),
            pltpu.VMEM((_HCH, _W), jnp.float32),
            pltpu.VMEM((_HCH, _W), jnp.float32),
            pltpu.SemaphoreType.DMA,
            pltpu.SemaphoreType.DMA,
            pltpu.SemaphoreType.DMA,
            pltpu.SemaphoreType.DMA,
        ],
    )
    return f(x, li)


def kernel(x):
    return _run(x, jnp.asarray(_LI_NP))
